# baseline (speedup 1.0000x reference)
import functools

import jax
import jax.numpy as jnp
from jax import lax
from jax.experimental import pallas as pl
from jax.experimental.pallas import tpu as pltpu

H = 128
NB = 4
DEPTH = 6


def _tables_body(emb_ref, wr_ref, wzx_ref, whx_ref, wox_ref, bz_ref, bh_ref,
                 bo_ref, tab_ref):
    emb = emb_ref[...]
    tab_ref[:, 0 * H:1 * H] = emb @ wr_ref[...]
    tab_ref[:, 1 * H:2 * H] = emb @ wzx_ref[...] + bz_ref[...]
    tab_ref[:, 2 * H:3 * H] = emb @ whx_ref[...] + bh_ref[...]
    tab_ref[:, 3 * H:4 * H] = emb @ wox_ref[...] + bo_ref[...]


def _make_tables(embedding, W_r, Wz_x, Wh_x, Wo_x, b_z, b_h, b_out):
    V = embedding.shape[0]
    return pl.pallas_call(
        _tables_body,
        out_shape=jax.ShapeDtypeStruct((V, 4 * H), jnp.float32),
    )(embedding, W_r.T, Wz_x.T, Wh_x.T, Wo_x.T,
      b_z[None, :], b_h[None, :], b_out[None, :])


def _gru_body(xz_ref, xh_ref, sumh_ref, sumg_ref, wzh_ref, whh_ref, ur_ref,
              bur_ref, hhu_ref, *, be):
    sumh = sumh_ref[...]
    z = jax.nn.sigmoid(xz_ref[...] + sumh @ wzh_ref[...])
    p = jnp.tanh(xh_ref[...] + sumg_ref[...] @ whh_ref[...])
    h = (1.0 - z) * sumh + z * p
    row = pl.program_id(0) * be + lax.broadcasted_iota(jnp.int32, (be, 1), 0)
    h = jnp.where(row > 0, h, 0.0)
    hhu_ref[:, :H] = h
    hhu_ref[:, H:] = h @ ur_ref[...] + bur_ref[...]


def _gru_dense(xz, xh, sumh, sumg, wzh, whh, ur_t, bur, *, be=1000):
    E = sumh.shape[0]
    grid = E // be
    bs_e = pl.BlockSpec((be, H), lambda i: (i, 0))
    bs_w = pl.BlockSpec((H, H), lambda i: (0, 0))
    bs_b = pl.BlockSpec((1, H), lambda i: (0, 0))
    return pl.pallas_call(
        functools.partial(_gru_body, be=be),
        grid=(grid,),
        in_specs=[bs_e, bs_e, bs_e, bs_e, bs_w, bs_w, bs_w, bs_b],
        out_specs=pl.BlockSpec((be, 2 * H), lambda i: (i, 0)),
        out_shape=jax.ShapeDtypeStruct((E, 2 * H), jnp.float32),
    )(xz, xh, sumh, sumg, wzh, whh, ur_t, bur[None, :])


def _out_body(eo_ref, msum_ref, woh_ref, nv_ref):
    nv_ref[...] = jnp.maximum(eo_ref[...] + msum_ref[...] @ woh_ref[...], 0.0)


def _node_out(eo_gath, msum, woh_t, *, bn=1000):
    N = msum.shape[0]
    bs_n = pl.BlockSpec((bn, H), lambda i: (i, 0))
    return pl.pallas_call(
        _out_body,
        grid=(N // bn,),
        in_specs=[bs_n, bs_n, pl.BlockSpec((H, H), lambda i: (0, 0))],
        out_specs=bs_n,
        out_shape=jax.ShapeDtypeStruct((N, H), jnp.float32),
    )(eo_gath, msum, woh_t)


def kernel(node_wid_list, edge_node_idx_list, node_message_graph,
           mess_adjacency_graph, scope, embedding, W_z, b_z, W_r, U_r, b_Ur,
           W_h, b_h, W_out, b_out):
    E = mess_adjacency_graph.shape[0]
    N = node_wid_list.shape[0]

    Wz_x, Wz_h = W_z[:, :H], W_z[:, H:]
    Wh_x, Wh_h = W_h[:, :H], W_h[:, H:]
    Wo_x, Wo_h = W_out[:, :H], W_out[:, H:]

    tab = _make_tables(embedding, W_r, Wz_x, Wh_x, Wo_x, b_z, b_h, b_out)

    widx = jnp.take(node_wid_list, edge_node_idx_list, axis=0)
    xtab = jnp.take(tab[:, :3 * H], widx, axis=0)
    xr, xz, xh = xtab[:, :H], xtab[:, H:2 * H], xtab[:, 2 * H:]

    hhu = jnp.concatenate(
        [jnp.zeros((E, H), jnp.float32),
         jnp.broadcast_to(b_Ur[None, :], (E, H))], axis=1)

    wzh_t, whh_t, ur_t, woh_t = Wz_h.T, Wh_h.T, U_r.T, Wo_h.T

    for _ in range(DEPTH):
        nei = jnp.take(hhu, mess_adjacency_graph, axis=0)
        h_nei, hu_nei = nei[:, :, :H], nei[:, :, H:]
        sumh = h_nei.sum(axis=1)
        r = jax.nn.sigmoid(xr[:, None, :] + hu_nei)
        sumg = (r * h_nei).sum(axis=1)
        hhu = _gru_dense(xz, xh, sumh, sumg, wzh_t, whh_t, ur_t, b_Ur)

    messages = hhu[:, :H]

    msum = jnp.take(messages, node_message_graph, axis=0).sum(axis=1)
    eo = jnp.take(tab[:, 3 * H:], node_wid_list, axis=0)
    node_vecs = _node_out(eo, msum, woh_t)
    tree_vecs = jnp.take(node_vecs, scope[:, 0], axis=0)
    return (tree_vecs, messages)


# speedup vs baseline: 1.6684x; 1.6684x over previous
import functools

import jax
import jax.numpy as jnp
from jax import lax
from jax.experimental import pallas as pl
from jax.experimental.pallas import tpu as pltpu
from jax.experimental.pallas import tpu_sc as plsc

H = 128
NB = 4
DEPTH = 6

_NC, _NS, _L = 2, 16, 16
_NW = _NC * _NS


def _sc_nei_body(hhu_hbm, xr_hbm, adjf_hbm, sumh_hbm, sumg_hbm,
                 idx_v, rows_v, xr_v, sumh_v, sumg_v, sem, *, ew, be):
    wid = lax.axis_index("s") * _NC + lax.axis_index("c")
    w0 = wid * ew
    nblk = ew // be

    def blk_body(b, carry):
        base = w0 + b * be
        pltpu.sync_copy(adjf_hbm.at[pl.ds(base * NB, be * NB)], idx_v)
        cp = pltpu.async_copy(hhu_hbm.at[idx_v], rows_v, sem)
        pltpu.sync_copy(xr_hbm.at[pl.ds(base, be)], xr_v)
        cp.wait()

        def edge_body(e, c2):
            r0 = e * NB
            for j in range(H // _L):
                o = j * _L
                xr16 = xr_v[e, pl.ds(o, _L)]
                s = jnp.zeros((_L,), jnp.float32)
                g = jnp.zeros((_L,), jnp.float32)
                for m in range(NB):
                    hv = rows_v[r0 + m, pl.ds(o, _L)]
                    hu = rows_v[r0 + m, pl.ds(H + o, _L)]
                    s = s + hv
                    sig = 1.0 / (1.0 + jnp.exp(-xr16 - hu))
                    g = g + sig * hv
                sumh_v[e, pl.ds(o, _L)] = s
                sumg_v[e, pl.ds(o, _L)] = g
            return c2

        lax.fori_loop(0, be, edge_body, 0)
        pltpu.sync_copy(sumh_v, sumh_hbm.at[pl.ds(base, be)])
        pltpu.sync_copy(sumg_v, sumg_hbm.at[pl.ds(base, be)])
        return carry

    lax.fori_loop(0, nblk, blk_body, 0)


def _sc_neighbor(hhu, xr, adjf, *, be=40):
    E = xr.shape[0]
    ew = E // _NW
    mesh = plsc.VectorSubcoreMesh(core_axis_name="c", subcore_axis_name="s")
    f = pl.kernel(
        functools.partial(_sc_nei_body, ew=ew, be=be),
        out_type=(jax.ShapeDtypeStruct((E, H), jnp.float32),
                  jax.ShapeDtypeStruct((E, H), jnp.float32)),
        mesh=mesh,
        scratch_types=[
            pltpu.VMEM((be * NB,), jnp.int32),
            pltpu.VMEM((be * NB, 2 * H), jnp.float32),
            pltpu.VMEM((be, H), jnp.float32),
            pltpu.VMEM((be, H), jnp.float32),
            pltpu.VMEM((be, H), jnp.float32),
            pltpu.SemaphoreType.DMA,
        ],
    )
    return f(hhu, xr, adjf)


def _tables_body(emb_ref, wr_ref, wzx_ref, whx_ref, wox_ref, bz_ref, bh_ref,
                 bo_ref, tab_ref):
    emb = emb_ref[...]
    tab_ref[:, 0 * H:1 * H] = emb @ wr_ref[...]
    tab_ref[:, 1 * H:2 * H] = emb @ wzx_ref[...] + bz_ref[...]
    tab_ref[:, 2 * H:3 * H] = emb @ whx_ref[...] + bh_ref[...]
    tab_ref[:, 3 * H:4 * H] = emb @ wox_ref[...] + bo_ref[...]


def _make_tables(embedding, W_r, Wz_x, Wh_x, Wo_x, b_z, b_h, b_out):
    V = embedding.shape[0]
    return pl.pallas_call(
        _tables_body,
        out_shape=jax.ShapeDtypeStruct((V, 4 * H), jnp.float32),
    )(embedding, W_r.T, Wz_x.T, Wh_x.T, Wo_x.T,
      b_z[None, :], b_h[None, :], b_out[None, :])


def _gru_body(xz_ref, xh_ref, sumh_ref, sumg_ref, wzh_ref, whh_ref, ur_ref,
              bur_ref, hhu_ref, *, be):
    sumh = sumh_ref[...]
    z = jax.nn.sigmoid(xz_ref[...] + sumh @ wzh_ref[...])
    p = jnp.tanh(xh_ref[...] + sumg_ref[...] @ whh_ref[...])
    h = (1.0 - z) * sumh + z * p
    row = pl.program_id(0) * be + lax.broadcasted_iota(jnp.int32, (be, 1), 0)
    h = jnp.where(row > 0, h, 0.0)
    hhu_ref[:, :H] = h
    hhu_ref[:, H:] = h @ ur_ref[...] + bur_ref[...]


def _gru_dense(xz, xh, sumh, sumg, wzh, whh, ur_t, bur, *, be=1000):
    E = sumh.shape[0]
    grid = E // be
    bs_e = pl.BlockSpec((be, H), lambda i: (i, 0))
    bs_w = pl.BlockSpec((H, H), lambda i: (0, 0))
    bs_b = pl.BlockSpec((1, H), lambda i: (0, 0))
    return pl.pallas_call(
        functools.partial(_gru_body, be=be),
        grid=(grid,),
        in_specs=[bs_e, bs_e, bs_e, bs_e, bs_w, bs_w, bs_w, bs_b],
        out_specs=pl.BlockSpec((be, 2 * H), lambda i: (i, 0)),
        out_shape=jax.ShapeDtypeStruct((E, 2 * H), jnp.float32),
    )(xz, xh, sumh, sumg, wzh, whh, ur_t, bur[None, :])


def _out_body(eo_ref, msum_ref, woh_ref, nv_ref):
    nv_ref[...] = jnp.maximum(eo_ref[...] + msum_ref[...] @ woh_ref[...], 0.0)


def _node_out(eo_gath, msum, woh_t, *, bn=1000):
    N = msum.shape[0]
    bs_n = pl.BlockSpec((bn, H), lambda i: (i, 0))
    return pl.pallas_call(
        _out_body,
        grid=(N // bn,),
        in_specs=[bs_n, bs_n, pl.BlockSpec((H, H), lambda i: (0, 0))],
        out_specs=bs_n,
        out_shape=jax.ShapeDtypeStruct((N, H), jnp.float32),
    )(eo_gath, msum, woh_t)


def kernel(node_wid_list, edge_node_idx_list, node_message_graph,
           mess_adjacency_graph, scope, embedding, W_z, b_z, W_r, U_r, b_Ur,
           W_h, b_h, W_out, b_out):
    E = mess_adjacency_graph.shape[0]
    N = node_wid_list.shape[0]

    Wz_x, Wz_h = W_z[:, :H], W_z[:, H:]
    Wh_x, Wh_h = W_h[:, :H], W_h[:, H:]
    Wo_x, Wo_h = W_out[:, :H], W_out[:, H:]

    tab = _make_tables(embedding, W_r, Wz_x, Wh_x, Wo_x, b_z, b_h, b_out)

    widx = jnp.take(node_wid_list, edge_node_idx_list, axis=0)
    xtab = jnp.take(tab[:, :3 * H], widx, axis=0)
    xr, xz, xh = xtab[:, :H], xtab[:, H:2 * H], xtab[:, 2 * H:]

    hhu = jnp.concatenate(
        [jnp.zeros((E, H), jnp.float32),
         jnp.broadcast_to(b_Ur[None, :], (E, H))], axis=1)

    wzh_t, whh_t, ur_t, woh_t = Wz_h.T, Wh_h.T, U_r.T, Wo_h.T
    adjf = mess_adjacency_graph.reshape(E * NB)

    for _ in range(DEPTH):
        sumh, sumg = _sc_neighbor(hhu, xr, adjf)
        hhu = _gru_dense(xz, xh, sumh, sumg, wzh_t, whh_t, ur_t, b_Ur)

    messages = hhu[:, :H]

    msum = jnp.take(messages, node_message_graph, axis=0).sum(axis=1)
    eo = jnp.take(tab[:, 3 * H:], node_wid_list, axis=0)
    node_vecs = _node_out(eo, msum, woh_t)
    tree_vecs = jnp.take(node_vecs, scope[:, 0], axis=0)
    return (tree_vecs, messages)


# speedup vs baseline: 2.8879x; 1.7310x over previous
import functools

import jax
import jax.numpy as jnp
from jax import lax
from jax.experimental import pallas as pl
from jax.experimental.pallas import tpu as pltpu
from jax.experimental.pallas import tpu_sc as plsc

H = 128
NB = 4
DEPTH = 6

_NC, _NS, _L = 2, 16, 16
_NW = _NC * _NS


def _sc_nei_body(heu_hbm, exr_hbm, adjf_hbm, sumh_hbm, sumg_hbm,
                 idx0, idx1, rows0, rows1, exr_v, sumh_v, sumg_v, sem0, sem1,
                 *, ew, be):
    wid = lax.axis_index("s") * _NC + lax.axis_index("c")
    w0 = wid * ew
    nblk = ew // be

    def fire(idx_v, rows_v, sem, base):
        pltpu.sync_copy(adjf_hbm.at[pl.ds(base * NB, be * NB)], idx_v)
        pltpu.async_copy(heu_hbm.at[idx_v], rows_v, sem)

    def compute(idx_v, rows_v, sem, base):
        pltpu.sync_copy(exr_hbm.at[pl.ds(base, be)], exr_v)
        pltpu.make_async_copy(heu_hbm.at[idx_v], rows_v, sem).wait()

        @plsc.parallel_loop(0, be, 1, unroll=2)
        def _edge(e):
            r0 = e * NB
            for j in range(H // _L):
                o = j * _L
                ex16 = exr_v[e, pl.ds(o, _L)]
                s = jnp.zeros((_L,), jnp.float32)
                g = jnp.zeros((_L,), jnp.float32)
                for m in range(NB):
                    hv = rows_v[r0 + m, pl.ds(o, _L)]
                    eu = rows_v[r0 + m, pl.ds(H + o, _L)]
                    s = s + hv
                    g = g + hv / (1.0 + ex16 * eu)
                sumh_v[e, pl.ds(o, _L)] = s
                sumg_v[e, pl.ds(o, _L)] = g

        pltpu.sync_copy(sumh_v, sumh_hbm.at[pl.ds(base, be)])
        pltpu.sync_copy(sumg_v, sumg_hbm.at[pl.ds(base, be)])

    fire(idx0, rows0, sem0, w0)

    def pair(i, carry):
        b0 = w0 + (2 * i) * be
        b1 = b0 + be
        fire(idx1, rows1, sem1, b1)
        compute(idx0, rows0, sem0, b0)
        fire(idx0, rows0, sem0, b1 + be)
        compute(idx1, rows1, sem1, b1)
        return carry

    lax.fori_loop(0, nblk // 2, pair, 0)
    compute(idx0, rows0, sem0, w0 + (nblk - 1) * be)


def _sc_neighbor(heu, exr, adjf, *, be=40):
    E = exr.shape[0]
    ew = E // _NW
    mesh = plsc.VectorSubcoreMesh(core_axis_name="c", subcore_axis_name="s")
    f = pl.kernel(
        functools.partial(_sc_nei_body, ew=ew, be=be),
        out_type=(jax.ShapeDtypeStruct((E, H), jnp.float32),
                  jax.ShapeDtypeStruct((E, H), jnp.float32)),
        mesh=mesh,
        scratch_types=[
            pltpu.VMEM((be * NB,), jnp.int32),
            pltpu.VMEM((be * NB,), jnp.int32),
            pltpu.VMEM((be * NB, 2 * H), jnp.float32),
            pltpu.VMEM((be * NB, 2 * H), jnp.float32),
            pltpu.VMEM((be, H), jnp.float32),
            pltpu.VMEM((be, H), jnp.float32),
            pltpu.VMEM((be, H), jnp.float32),
            pltpu.SemaphoreType.DMA,
            pltpu.SemaphoreType.DMA,
        ],
    )
    return f(heu, exr, adjf)


def _tables_body(emb_ref, wr_ref, wzx_ref, whx_ref, wox_ref, bz_ref, bh_ref,
                 bo_ref, tab_ref):
    emb = emb_ref[...]
    tab_ref[:, 0 * H:1 * H] = jnp.exp(-(emb @ wr_ref[...]))
    tab_ref[:, 1 * H:2 * H] = emb @ wzx_ref[...] + bz_ref[...]
    tab_ref[:, 2 * H:3 * H] = emb @ whx_ref[...] + bh_ref[...]
    tab_ref[:, 3 * H:4 * H] = emb @ wox_ref[...] + bo_ref[...]


def _make_tables(embedding, W_r, Wz_x, Wh_x, Wo_x, b_z, b_h, b_out):
    V = embedding.shape[0]
    return pl.pallas_call(
        _tables_body,
        out_shape=jax.ShapeDtypeStruct((V, 4 * H), jnp.float32),
    )(embedding, W_r.T, Wz_x.T, Wh_x.T, Wo_x.T,
      b_z[None, :], b_h[None, :], b_out[None, :])


def _gru_body(xz_ref, xh_ref, sumh_ref, sumg_ref, wzh_ref, whh_ref, ur_ref,
              bur_ref, hhu_ref, *, be):
    sumh = sumh_ref[...]
    z = jax.nn.sigmoid(xz_ref[...] + sumh @ wzh_ref[...])
    p = jnp.tanh(xh_ref[...] + sumg_ref[...] @ whh_ref[...])
    h = (1.0 - z) * sumh + z * p
    row = pl.program_id(0) * be + lax.broadcasted_iota(jnp.int32, (be, 1), 0)
    h = jnp.where(row > 0, h, 0.0)
    hhu_ref[:, :H] = h
    hhu_ref[:, H:] = jnp.exp(-(h @ ur_ref[...] + bur_ref[...]))


def _gru_dense(xz, xh, sumh, sumg, wzh, whh, ur_t, bur, *, be=1000):
    E = sumh.shape[0]
    grid = E // be
    bs_e = pl.BlockSpec((be, H), lambda i: (i, 0))
    bs_w = pl.BlockSpec((H, H), lambda i: (0, 0))
    bs_b = pl.BlockSpec((1, H), lambda i: (0, 0))
    return pl.pallas_call(
        functools.partial(_gru_body, be=be),
        grid=(grid,),
        in_specs=[bs_e, bs_e, bs_e, bs_e, bs_w, bs_w, bs_w, bs_b],
        out_specs=pl.BlockSpec((be, 2 * H), lambda i: (i, 0)),
        out_shape=jax.ShapeDtypeStruct((E, 2 * H), jnp.float32),
    )(xz, xh, sumh, sumg, wzh, whh, ur_t, bur[None, :])


def _out_body(eo_ref, msum_ref, woh_ref, nv_ref):
    nv_ref[...] = jnp.maximum(eo_ref[...] + msum_ref[...] @ woh_ref[...], 0.0)


def _node_out(eo_gath, msum, woh_t, *, bn=1000):
    N = msum.shape[0]
    bs_n = pl.BlockSpec((bn, H), lambda i: (i, 0))
    return pl.pallas_call(
        _out_body,
        grid=(N // bn,),
        in_specs=[bs_n, bs_n, pl.BlockSpec((H, H), lambda i: (0, 0))],
        out_specs=bs_n,
        out_shape=jax.ShapeDtypeStruct((N, H), jnp.float32),
    )(eo_gath, msum, woh_t)


def kernel(node_wid_list, edge_node_idx_list, node_message_graph,
           mess_adjacency_graph, scope, embedding, W_z, b_z, W_r, U_r, b_Ur,
           W_h, b_h, W_out, b_out):
    E = mess_adjacency_graph.shape[0]
    N = node_wid_list.shape[0]

    Wz_x, Wz_h = W_z[:, :H], W_z[:, H:]
    Wh_x, Wh_h = W_h[:, :H], W_h[:, H:]
    Wo_x, Wo_h = W_out[:, :H], W_out[:, H:]

    tab = _make_tables(embedding, W_r, Wz_x, Wh_x, Wo_x, b_z, b_h, b_out)

    widx = jnp.take(node_wid_list, edge_node_idx_list, axis=0)
    xtab = jnp.take(tab[:, :3 * H], widx, axis=0)
    exr, xz, xh = xtab[:, :H], xtab[:, H:2 * H], xtab[:, 2 * H:]

    hhu = jnp.concatenate(
        [jnp.zeros((E, H), jnp.float32),
         jnp.broadcast_to(jnp.exp(-b_Ur)[None, :], (E, H))], axis=1)

    wzh_t, whh_t, ur_t, woh_t = Wz_h.T, Wh_h.T, U_r.T, Wo_h.T
    adjf = mess_adjacency_graph.reshape(E * NB)

    for _ in range(DEPTH):
        sumh, sumg = _sc_neighbor(hhu, exr, adjf)
        hhu = _gru_dense(xz, xh, sumh, sumg, wzh_t, whh_t, ur_t, b_Ur)

    messages = hhu[:, :H]

    msum = jnp.take(messages, node_message_graph, axis=0).sum(axis=1)
    eo = jnp.take(tab[:, 3 * H:], node_wid_list, axis=0)
    node_vecs = _node_out(eo, msum, woh_t)
    tree_vecs = jnp.take(node_vecs, scope[:, 0], axis=0)
    return (tree_vecs, messages)


# speedup vs baseline: 3.3186x; 1.1491x over previous
import functools

import jax
import jax.numpy as jnp
from jax import lax
from jax.experimental import pallas as pl
from jax.experimental.pallas import tpu as pltpu
from jax.experimental.pallas import tpu_sc as plsc

H = 128
NB = 4
DEPTH = 6

_NC, _NS, _L = 2, 16, 16
_NW = _NC * _NS


def _sc_invar_body(widx_hbm, tabe_hbm, tabzh_hbm, exr_hbm, xzh_hbm,
                   widx_v, erows_v, zrows_v, sem, *, ew, be):
    wid = lax.axis_index("s") * _NC + lax.axis_index("c")
    w0 = wid * ew
    nblk = ew // be
    tail = ew - nblk * be

    def do_block(base, cnt):
        pltpu.sync_copy(widx_hbm.at[pl.ds(base, cnt)],
                        widx_v.at[pl.ds(0, cnt)])
        pltpu.async_copy(tabe_hbm.at[widx_v.at[pl.ds(0, cnt)]],
                         erows_v.at[pl.ds(0, cnt)], sem).wait()
        pltpu.async_copy(tabzh_hbm.at[widx_v.at[pl.ds(0, cnt)]],
                         zrows_v.at[pl.ds(0, cnt)], sem).wait()
        pltpu.sync_copy(erows_v.at[pl.ds(0, cnt)], exr_hbm.at[pl.ds(base, cnt)])
        pltpu.sync_copy(zrows_v.at[pl.ds(0, cnt)], xzh_hbm.at[pl.ds(base, cnt)])

    def blk(b, c):
        do_block(w0 + b * be, be)
        return c

    lax.fori_loop(0, nblk, blk, 0)
    if tail:
        do_block(w0 + nblk * be, tail)


def _sc_invariants(widx, tabe, tabzh, *, be=128):
    E = widx.shape[0]
    ew = E // _NW
    mesh = plsc.VectorSubcoreMesh(core_axis_name="c", subcore_axis_name="s")
    f = pl.kernel(
        functools.partial(_sc_invar_body, ew=ew, be=be),
        out_type=(jax.ShapeDtypeStruct((E, H), jnp.float32),
                  jax.ShapeDtypeStruct((E, 2 * H), jnp.float32)),
        mesh=mesh,
        scratch_types=[
            pltpu.VMEM((be,), jnp.int32),
            pltpu.VMEM((be, H), jnp.float32),
            pltpu.VMEM((be, 2 * H), jnp.float32),
            pltpu.SemaphoreType.DMA,
        ],
    )
    return f(widx, tabe, tabzh)


def _sc_root_body(snmgf_hbm, swid_hbm, msgs_hbm, tabo_hbm, msum_hbm, eo_hbm,
                  idx_v, rows_v, widx_v, orows_v, msum_v, sem, *, bw):
    wid = lax.axis_index("s") * _NC + lax.axis_index("c")
    base = wid * bw
    pltpu.sync_copy(snmgf_hbm.at[pl.ds(base * NB, bw * NB)], idx_v)
    cp = pltpu.async_copy(msgs_hbm.at[idx_v], rows_v, sem)
    pltpu.sync_copy(swid_hbm.at[pl.ds(base, bw)], widx_v)
    cp.wait()
    pltpu.async_copy(tabo_hbm.at[widx_v], orows_v, sem).wait()
    for nloc in range(bw):
        r0 = nloc * NB
        for j in range(H // _L):
            o = j * _L
            s = rows_v[r0, pl.ds(o, _L)]
            for m in range(1, NB):
                s = s + rows_v[r0 + m, pl.ds(o, _L)]
            msum_v[nloc, pl.ds(o, _L)] = s
    pltpu.sync_copy(msum_v, msum_hbm.at[pl.ds(base, bw)])
    pltpu.sync_copy(orows_v, eo_hbm.at[pl.ds(base, bw)])


def _sc_root(snmgf, swid, msgs, tabo):
    B = swid.shape[0]
    bw = B // _NW
    mesh = plsc.VectorSubcoreMesh(core_axis_name="c", subcore_axis_name="s")
    f = pl.kernel(
        functools.partial(_sc_root_body, bw=bw),
        out_type=(jax.ShapeDtypeStruct((B, H), jnp.float32),
                  jax.ShapeDtypeStruct((B, H), jnp.float32)),
        mesh=mesh,
        scratch_types=[
            pltpu.VMEM((bw * NB,), jnp.int32),
            pltpu.VMEM((bw * NB, H), jnp.float32),
            pltpu.VMEM((bw,), jnp.int32),
            pltpu.VMEM((bw, H), jnp.float32),
            pltpu.VMEM((bw, H), jnp.float32),
            pltpu.SemaphoreType.DMA,
        ],
    )
    return f(snmgf, swid, msgs, tabo)


def _sc_nei_body(heu_hbm, exr_hbm, adjf_hbm, sumh_hbm, sumg_hbm,
                 idx0, idx1, rows0, rows1, exr_v, sumh_v, sumg_v, sem0, sem1,
                 *, ew, be):
    wid = lax.axis_index("s") * _NC + lax.axis_index("c")
    w0 = wid * ew
    nblk = ew // be

    def fire(idx_v, rows_v, sem, base):
        pltpu.sync_copy(adjf_hbm.at[pl.ds(base * NB, be * NB)], idx_v)
        pltpu.async_copy(heu_hbm.at[idx_v], rows_v, sem)

    def compute(idx_v, rows_v, sem, base):
        pltpu.sync_copy(exr_hbm.at[pl.ds(base, be)], exr_v)
        pltpu.make_async_copy(heu_hbm.at[idx_v], rows_v, sem).wait()

        @plsc.parallel_loop(0, be, 1, unroll=2)
        def _edge(e):
            r0 = e * NB
            for j in range(H // _L):
                o = j * _L
                ex16 = exr_v[e, pl.ds(o, _L)]
                s = jnp.zeros((_L,), jnp.float32)
                g = jnp.zeros((_L,), jnp.float32)
                for m in range(NB):
                    hv = rows_v[r0 + m, pl.ds(o, _L)]
                    eu = rows_v[r0 + m, pl.ds(H + o, _L)]
                    s = s + hv
                    g = g + hv / (1.0 + ex16 * eu)
                sumh_v[e, pl.ds(o, _L)] = s
                sumg_v[e, pl.ds(o, _L)] = g

        pltpu.sync_copy(sumh_v, sumh_hbm.at[pl.ds(base, be)])
        pltpu.sync_copy(sumg_v, sumg_hbm.at[pl.ds(base, be)])

    fire(idx0, rows0, sem0, w0)

    def pair(i, carry):
        b0 = w0 + (2 * i) * be
        b1 = b0 + be
        fire(idx1, rows1, sem1, b1)
        compute(idx0, rows0, sem0, b0)
        fire(idx0, rows0, sem0, b1 + be)
        compute(idx1, rows1, sem1, b1)
        return carry

    lax.fori_loop(0, nblk // 2, pair, 0)
    compute(idx0, rows0, sem0, w0 + (nblk - 1) * be)


def _sc_neighbor(heu, exr, adjf, *, be=40):
    E = exr.shape[0]
    ew = E // _NW
    mesh = plsc.VectorSubcoreMesh(core_axis_name="c", subcore_axis_name="s")
    f = pl.kernel(
        functools.partial(_sc_nei_body, ew=ew, be=be),
        out_type=(jax.ShapeDtypeStruct((E, H), jnp.float32),
                  jax.ShapeDtypeStruct((E, H), jnp.float32)),
        mesh=mesh,
        scratch_types=[
            pltpu.VMEM((be * NB,), jnp.int32),
            pltpu.VMEM((be * NB,), jnp.int32),
            pltpu.VMEM((be * NB, 2 * H), jnp.float32),
            pltpu.VMEM((be * NB, 2 * H), jnp.float32),
            pltpu.VMEM((be, H), jnp.float32),
            pltpu.VMEM((be, H), jnp.float32),
            pltpu.VMEM((be, H), jnp.float32),
            pltpu.SemaphoreType.DMA,
            pltpu.SemaphoreType.DMA,
        ],
    )
    return f(heu, exr, adjf)


def _tables_body(emb_ref, wr_ref, wzx_ref, whx_ref, wox_ref, bz_ref, bh_ref,
                 bo_ref, tabe_ref, tabzh_ref, tabo_ref):
    emb = emb_ref[...]
    tabe_ref[...] = jnp.exp(-(emb @ wr_ref[...]))
    tabzh_ref[:, :H] = emb @ wzx_ref[...] + bz_ref[...]
    tabzh_ref[:, H:] = emb @ whx_ref[...] + bh_ref[...]
    tabo_ref[...] = emb @ wox_ref[...] + bo_ref[...]


def _make_tables(embedding, W_r, Wz_x, Wh_x, Wo_x, b_z, b_h, b_out):
    V = embedding.shape[0]
    return pl.pallas_call(
        _tables_body,
        out_shape=(jax.ShapeDtypeStruct((V, H), jnp.float32),
                   jax.ShapeDtypeStruct((V, 2 * H), jnp.float32),
                   jax.ShapeDtypeStruct((V, H), jnp.float32)),
    )(embedding, W_r.T, Wz_x.T, Wh_x.T, Wo_x.T,
      b_z[None, :], b_h[None, :], b_out[None, :])


def _gru_h(xzh_ref, sumh_ref, sumg_ref, wzh_ref, whh_ref, be):
    sumh = sumh_ref[...]
    z = jax.nn.sigmoid(xzh_ref[:, :H] + sumh @ wzh_ref[...])
    p = jnp.tanh(xzh_ref[:, H:] + sumg_ref[...] @ whh_ref[...])
    h = (1.0 - z) * sumh + z * p
    row = pl.program_id(0) * be + lax.broadcasted_iota(jnp.int32, (be, 1), 0)
    return jnp.where(row > 0, h, 0.0)


def _gru_body(xzh_ref, sumh_ref, sumg_ref, wzh_ref, whh_ref, ur_ref,
              bur_ref, hhu_ref, *, be):
    h = _gru_h(xzh_ref, sumh_ref, sumg_ref, wzh_ref, whh_ref, be)
    hhu_ref[:, :H] = h
    hhu_ref[:, H:] = jnp.exp(-(h @ ur_ref[...] + bur_ref[...]))


def _gru_last_body(xzh_ref, sumh_ref, sumg_ref, wzh_ref, whh_ref, h_ref, *, be):
    h_ref[...] = _gru_h(xzh_ref, sumh_ref, sumg_ref, wzh_ref, whh_ref, be)


def _gru_dense(xzh, sumh, sumg, wzh, whh, ur_t, bur, *, be=1000):
    E = sumh.shape[0]
    bs_e = pl.BlockSpec((be, H), lambda i: (i, 0))
    bs_w = pl.BlockSpec((H, H), lambda i: (0, 0))
    return pl.pallas_call(
        functools.partial(_gru_body, be=be),
        grid=(E // be,),
        in_specs=[pl.BlockSpec((be, 2 * H), lambda i: (i, 0)),
                  bs_e, bs_e, bs_w, bs_w, bs_w,
                  pl.BlockSpec((1, H), lambda i: (0, 0))],
        out_specs=pl.BlockSpec((be, 2 * H), lambda i: (i, 0)),
        out_shape=jax.ShapeDtypeStruct((E, 2 * H), jnp.float32),
    )(xzh, sumh, sumg, wzh, whh, ur_t, bur[None, :])


def _gru_dense_last(xzh, sumh, sumg, wzh, whh, *, be=1000):
    E = sumh.shape[0]
    bs_e = pl.BlockSpec((be, H), lambda i: (i, 0))
    bs_w = pl.BlockSpec((H, H), lambda i: (0, 0))
    return pl.pallas_call(
        functools.partial(_gru_last_body, be=be),
        grid=(E // be,),
        in_specs=[pl.BlockSpec((be, 2 * H), lambda i: (i, 0)),
                  bs_e, bs_e, bs_w, bs_w],
        out_specs=bs_e,
        out_shape=jax.ShapeDtypeStruct((E, H), jnp.float32),
    )(xzh, sumh, sumg, wzh, whh)


def _root_body(eo_ref, msum_ref, woh_ref, tv_ref):
    tv_ref[...] = jnp.maximum(eo_ref[...] + msum_ref[...] @ woh_ref[...], 0.0)


def _root_out(eo, msum, woh_t):
    B = msum.shape[0]
    return pl.pallas_call(
        _root_body,
        out_shape=jax.ShapeDtypeStruct((B, H), jnp.float32),
    )(eo, msum, woh_t)


def kernel(node_wid_list, edge_node_idx_list, node_message_graph,
           mess_adjacency_graph, scope, embedding, W_z, b_z, W_r, U_r, b_Ur,
           W_h, b_h, W_out, b_out):
    E = mess_adjacency_graph.shape[0]
    N = node_wid_list.shape[0]

    Wz_x, Wz_h = W_z[:, :H], W_z[:, H:]
    Wh_x, Wh_h = W_h[:, :H], W_h[:, H:]
    Wo_x, Wo_h = W_out[:, :H], W_out[:, H:]

    tabe, tabzh, tabo = _make_tables(embedding, W_r, Wz_x, Wh_x, Wo_x,
                                     b_z, b_h, b_out)

    widx = jnp.take(node_wid_list, edge_node_idx_list, axis=0)
    exr, xzh = _sc_invariants(widx, tabe, tabzh)

    hhu = jnp.concatenate(
        [jnp.zeros((E, H), jnp.float32),
         jnp.broadcast_to(jnp.exp(-b_Ur)[None, :], (E, H))], axis=1)

    wzh_t, whh_t, ur_t, woh_t = Wz_h.T, Wh_h.T, U_r.T, Wo_h.T
    adjf = mess_adjacency_graph.reshape(E * NB)

    for it in range(DEPTH):
        sumh, sumg = _sc_neighbor(hhu, exr, adjf)
        if it < DEPTH - 1:
            hhu = _gru_dense(xzh, sumh, sumg, wzh_t, whh_t, ur_t, b_Ur)
        else:
            messages = _gru_dense_last(xzh, sumh, sumg, wzh_t, whh_t)

    scope0 = scope[:, 0]
    swid = jnp.take(node_wid_list, scope0, axis=0)
    snmgf = jnp.take(node_message_graph, scope0, axis=0).reshape(-1)
    msum, eo = _sc_root(snmgf, swid, messages, tabo)
    tree_vecs = _root_out(eo, msum, woh_t)
    return (tree_vecs, messages)
